# baseline (speedup 1.0000x reference)
import jax
import jax.numpy as jnp
from jax import lax
from jax.experimental import pallas as pl
from jax.experimental.pallas import tpu as pltpu
from jax.experimental.pallas import tpu_sc as plsc

NC, NS, L = 2, 16, 16


def _tables_body(x_ref, pos_ref, wsrc_ref, wdst_ref, wlin_ref, wpos_ref,
                 bpos_ref, ta_ref, tb_ref):
    x = x_ref[...]
    posw = jnp.dot(pos_ref[...], wpos_ref[...],
                   preferred_element_type=jnp.float32)
    bp = bpos_ref[...]
    t1 = jnp.dot(x, wsrc_ref[...],
                 preferred_element_type=jnp.float32) + posw + bp
    t2 = jnp.dot(x, wdst_ref[...],
                 preferred_element_type=jnp.float32) + posw
    t3 = jnp.dot(x, wlin_ref[...],
                 preferred_element_type=jnp.float32) - posw
    t4 = posw + bp
    ta_ref[...] = jnp.concatenate([t1, t4], axis=1)
    tb_ref[...] = jnp.concatenate([t2, t3], axis=1)


def _make_tables(x, pos, W_src, W_dst, W_lin, W_pos, b_pos):
    n, d = x.shape
    blk = 1000
    nb = n // blk
    return pl.pallas_call(
        _tables_body,
        grid=(nb,),
        in_specs=[
            pl.BlockSpec((blk, d), lambda i: (i, 0)),
            pl.BlockSpec((blk, 3), lambda i: (i, 0)),
            pl.BlockSpec((d, d), lambda i: (0, 0)),
            pl.BlockSpec((d, d), lambda i: (0, 0)),
            pl.BlockSpec((d, d), lambda i: (0, 0)),
            pl.BlockSpec((3, d), lambda i: (0, 0)),
            pl.BlockSpec((1, d), lambda i: (0, 0)),
        ],
        out_specs=[pl.BlockSpec((blk, 2 * d), lambda i: (i, 0))] * 2,
        out_shape=[jax.ShapeDtypeStruct((n, 2 * d), jnp.float32)] * 2,
    )(x, pos, W_src, W_dst, W_lin, W_pos, b_pos.reshape(1, d))


def _alpha_sc_body(ta_hbm, tb_hbm, rows_hbm, cols_hbm, alpha_hbm, val_hbm,
                   rowb0, rowb1, colb0, colb1, tab0, tab1, tbb0, tbb1,
                   isem0, isem1, gsem0, gsem1, wsem0, wsem1):
    e_total = alpha_hbm.shape[0]
    nchunks = e_total // 80
    iters = nchunks // (NC * NS)
    cc = lax.axis_index("c")
    ss = lax.axis_index("s")
    wid = cc * NS + ss
    rowb = (rowb0, rowb1)
    colb = (colb0, colb1)
    tab = (tab0, tab1)
    tbb = (tbb0, tbb1)
    isem = (isem0, isem1)
    gsem = (gsem0, gsem1)
    wsem = (wsem0, wsem1)

    def e_of(k):
        return (wid + k * NC * NS) * 80

    def idx_copies(k, p):
        e0 = e_of(k)
        return (pltpu.make_async_copy(rows_hbm.at[pl.ds(e0, 80)], rowb[p],
                                      isem[p]),
                pltpu.make_async_copy(cols_hbm.at[pl.ds(e0, 80)], colb[p],
                                      isem[p]))

    def gather_copies(p):
        return (pltpu.make_async_copy(ta_hbm.at[rowb[p]], tab[p], gsem[p]),
                pltpu.make_async_copy(tb_hbm.at[colb[p]], tbb[p], gsem[p]))

    def write_copies(k, p):
        e0 = e_of(k)
        return (
            pltpu.make_async_copy(tab[p].at[pl.ds(0, 80), pl.ds(0, 128)],
                                  alpha_hbm.at[pl.ds(e0, 80)], wsem[p]),
            pltpu.make_async_copy(tab[p].at[pl.ds(0, 80), pl.ds(128, 128)],
                                  val_hbm.at[pl.ds(e0, 80)], wsem[p]))

    def compute(p):
        t, u = tab[p], tbb[p]

        def row_body(r, c2):
            for l in range(8):
                sl = pl.ds(l * 16, 16)
                t[r, sl] = t[r, sl] - u[r, sl]
            for l in range(8, 16):
                sl = pl.ds(l * 16, 16)
                t[r, sl] = t[r, sl] + u[r, sl]
            return c2
        lax.fori_loop(0, 80, row_body, 0, unroll=4)

    for d in idx_copies(0, 0):
        d.start()
    for d in idx_copies(1, 1):
        d.start()
    for d in idx_copies(0, 0):
        d.wait()
    for d in gather_copies(0):
        d.start()

    def step(k, p, k_is_first):
        p1 = 1 - p
        @pl.when(k + 1 < iters)
        def _():
            for d in idx_copies(k + 1, p1):
                d.wait()
            if not k_is_first:
                for d in write_copies(k - 1, p1):
                    d.wait()
            for d in gather_copies(p1):
                d.start()

        @pl.when(jnp.logical_and(k + 1 >= iters, k >= 1))
        def _():
            for d in write_copies(k - 1, p1):
                d.wait()
        for d in gather_copies(p):
            d.wait()
        @pl.when(k + 2 < iters)
        def _():
            for d in idx_copies(k + 2, p):
                d.start()
        compute(p)
        for d in write_copies(k, p):
            d.start()

    step(jnp.int32(0), 0, True)

    def pair(i, carry):
        step(2 * i + 1, 1, False)
        step(2 * i + 2, 0, False)
        return carry
    lax.fori_loop(0, (iters - 1) // 2, pair, 0)
    for d in write_copies(iters - 1, 0):
        d.wait()


def _alpha_gather(ta, tb, rows1d, cols1d, e_total):
    mesh = plsc.VectorSubcoreMesh(core_axis_name="c", subcore_axis_name="s")
    return pl.kernel(
        _alpha_sc_body,
        out_type=[
            jax.ShapeDtypeStruct((e_total, 128), jnp.float32),
            jax.ShapeDtypeStruct((e_total, 128), jnp.float32),
        ],
        mesh=mesh,
        scratch_types=(
            [pltpu.VMEM((80,), jnp.int32)] * 4 +
            [pltpu.VMEM((80, 256), jnp.float32)] * 4 +
            [pltpu.SemaphoreType.DMA] * 6
        ),
    )(ta, tb, rows1d, cols1d)


def _mlp_body(a_ref, wa1_ref, ba1_ref, wa2_ref, ba2_ref, a2_ref, mx_ref):
    h = jnp.maximum(
        jnp.dot(a_ref[...], wa1_ref[...],
                preferred_element_type=jnp.float32) + ba1_ref[...], 0.0)
    a2 = jnp.dot(h, wa2_ref[...],
                 preferred_element_type=jnp.float32) + ba2_ref[...]
    a2_ref[...] = a2
    mx_ref[...] = jnp.max(a2, axis=0, keepdims=True)[None]


def _edge_mlp(alpha, W_a1, b_a1, W_a2, b_a2):
    e_total, d = alpha.shape
    blk = 2000
    nb = e_total // blk
    a2, mx = pl.pallas_call(
        _mlp_body,
        grid=(nb,),
        in_specs=[
            pl.BlockSpec((blk, d), lambda i: (i, 0)),
            pl.BlockSpec((d, d), lambda i: (0, 0)),
            pl.BlockSpec((1, d), lambda i: (0, 0)),
            pl.BlockSpec((d, d), lambda i: (0, 0)),
            pl.BlockSpec((1, d), lambda i: (0, 0)),
        ],
        out_specs=[
            pl.BlockSpec((blk, d), lambda i: (i, 0)),
            pl.BlockSpec((1, 1, d), lambda i: (i, 0, 0)),
        ],
        out_shape=[
            jax.ShapeDtypeStruct((e_total, d), jnp.float32),
            jax.ShapeDtypeStruct((nb, 1, d), jnp.float32),
        ],
    )(alpha, W_a1, b_a1.reshape(1, d), W_a2, b_a2.reshape(1, d))
    return a2, mx


def _agg_sc_body(a2_hbm, val_hbm, rows_hbm, cvec_hbm, zeros_hbm,
                 s_hbm, o_hbm, rb0, rb1, a2b0, a2b1, vb0, vb1, cbuf,
                 acc_sp, lsem0, lsem1, ssem0, ssem1):
    n = s_hbm.shape[0]
    e_total = a2_hbm.shape[0]
    per_sub = e_total // NS
    iters = per_sub // 80
    cc = lax.axis_index("c")
    ss = lax.axis_index("s")
    rb = (rb0, rb1)
    a2b = (a2b0, a2b1)
    vb = (vb0, vb1)
    lsem = (lsem0, lsem1)
    ssem = (ssem0, ssem1)

    zbase = ((ss * (n // NS)) // 8) * 8
    pltpu.sync_copy(zeros_hbm, acc_sp.at[pl.ds(zbase, 632)])
    pltpu.sync_copy(cvec_hbm, cbuf)
    plsc.subcore_barrier()
    cv = cbuf[...]

    def load_copies(k, p):
        e0 = ss * per_sub + k * 80
        out = [pltpu.make_async_copy(rows_hbm.at[pl.ds(e0, 80)], rb[p],
                                     lsem[p]),
               pltpu.make_async_copy(a2_hbm.at[pl.ds(e0, 80)], a2b[p],
                                     lsem[p])]
        return out

    def val_copy(k, p):
        e0 = ss * per_sub + k * 80
        return pltpu.make_async_copy(val_hbm.at[pl.ds(e0, 80)], vb[p],
                                     lsem[p])

    def scat_copy(p, src):
        return pltpu.make_async_copy(src, acc_sp.at[rb[p]], ssem[p])

    def issue_loads(k, p):
        for d in load_copies(k, p):
            d.start()

        @pl.when(cc == 1)
        def _():
            val_copy(k, p).start()

    def wait_loads(k, p):
        for d in load_copies(k, p):
            d.wait()

        @pl.when(cc == 1)
        def _():
            val_copy(k, p).wait()

    def step(k, p, k_is_first):
        p1 = 1 - p
        if not k_is_first:
            @pl.when(cc == 0)
            def _():
                scat_copy(p1, a2b[p1]).wait()

            @pl.when(cc == 1)
            def _():
                scat_copy(p1, vb[p1]).wait()
        @pl.when(k + 1 < iters)
        def _():
            issue_loads(k + 1, p1)
        wait_loads(k, p)

        @pl.when(cc == 1)
        def _():
            def row_body1(r, c2):
                for l in range(8):
                    sl = pl.ds(l * 16, 16)
                    e = jnp.exp(a2b[p][r, sl] - cv)
                    vb[p][r, sl] = e * vb[p][r, sl]
                return c2
            lax.fori_loop(0, 80, row_body1, 0, unroll=4)
            scat_copy(p, vb[p]).start(add=True)

        @pl.when(cc == 0)
        def _():
            def row_body0(r, c2):
                for l in range(8):
                    sl = pl.ds(l * 16, 16)
                    a2b[p][r, sl] = jnp.exp(a2b[p][r, sl] - cv)
                return c2
            lax.fori_loop(0, 80, row_body0, 0, unroll=4)
            scat_copy(p, a2b[p]).start(add=True)

    issue_loads(jnp.int32(0), 0)
    step(jnp.int32(0), 0, True)

    def pair(i, carry):
        step(2 * i + 1, 1, False)
        step(2 * i + 2, 0, False)
        return carry
    lax.fori_loop(0, (iters - 2) // 2, pair, 0)
    step(jnp.int32(iters - 1), 1, False)
    @pl.when(cc == 0)
    def _():
        scat_copy(1, a2b[1]).wait()

    @pl.when(cc == 1)
    def _():
        scat_copy(1, vb[1]).wait()
    plsc.subcore_barrier()

    @pl.when(cc == 0)
    def _():
        pltpu.sync_copy(acc_sp.at[pl.ds(zbase, 632)],
                        s_hbm.at[pl.ds(zbase, 632)])

    @pl.when(cc == 1)
    def _():
        pltpu.sync_copy(acc_sp.at[pl.ds(zbase, 632)],
                        o_hbm.at[pl.ds(zbase, 632)])


def _softmax_agg(a2, val, rows1d, cvec, zeros, n):
    mesh = plsc.VectorSubcoreMesh(core_axis_name="c", subcore_axis_name="s")
    return pl.kernel(
        _agg_sc_body,
        out_type=[
            jax.ShapeDtypeStruct((n, 128), jnp.float32),
            jax.ShapeDtypeStruct((n, 128), jnp.float32),
        ],
        mesh=mesh,
        scratch_types=(
            [pltpu.VMEM((80,), jnp.int32)] * 2 +
            [pltpu.VMEM((80, 128), jnp.float32)] * 4 +
            [
                pltpu.VMEM((16,), jnp.float32),
                pltpu.VMEM_SHARED((n, 128), jnp.float32),
            ] +
            [pltpu.SemaphoreType.DMA] * 4
        ),
    )(a2, val, rows1d, cvec, zeros)


def _div_body(o_ref, s_ref, out_ref):
    out_ref[...] = o_ref[...] / (s_ref[...] + 1e-12)


def _divide(o_arr, s_arr):
    n, d = o_arr.shape
    blk = 1000
    return pl.pallas_call(
        _div_body,
        grid=(n // blk,),
        in_specs=[pl.BlockSpec((blk, d), lambda i: (i, 0))] * 2,
        out_specs=pl.BlockSpec((blk, d), lambda i: (i, 0)),
        out_shape=jax.ShapeDtypeStruct((n, d), jnp.float32),
    )(o_arr, s_arr)


def kernel(x, pos, edge_index, W_lin, W_src, W_dst, W_pos, b_pos,
           W_a1, b_a1, W_a2, b_a2):
    n = x.shape[0]
    e_total = edge_index.shape[0]
    rows = edge_index[:, 0]
    cols = edge_index[:, 1]

    ta, tb = _make_tables(x, pos, W_src, W_dst, W_lin, W_pos, b_pos)
    alpha, val = _alpha_gather(ta, tb, rows, cols, e_total)
    a2, mx = _edge_mlp(alpha, W_a1, b_a1, W_a2, b_a2)
    c = jnp.max(mx)
    cvec = jnp.full((16,), c, jnp.float32)
    zeros = jnp.zeros((632, 128), jnp.float32)
    s_arr, o_arr = _softmax_agg(a2, val, rows, cvec, zeros, n)
    return _divide(o_arr, s_arr)


# speedup vs baseline: 3.1585x; 3.1585x over previous
import jax
import jax.numpy as jnp
from jax import lax
from jax.experimental import pallas as pl
from jax.experimental.pallas import tpu as pltpu
from jax.experimental.pallas import tpu_sc as plsc

NC, NS, L = 2, 16, 16


def _tables_body(x_ref, pos_ref, wsrc_ref, wdst_ref, wlin_ref, wpos_ref,
                 bpos_ref, ta_ref, tb_ref):
    x = x_ref[...]
    posw = jnp.dot(pos_ref[...], wpos_ref[...],
                   preferred_element_type=jnp.float32)
    bp = bpos_ref[...]
    t1 = jnp.dot(x, wsrc_ref[...],
                 preferred_element_type=jnp.float32) + posw + bp
    t2 = jnp.dot(x, wdst_ref[...],
                 preferred_element_type=jnp.float32) + posw
    t3 = jnp.dot(x, wlin_ref[...],
                 preferred_element_type=jnp.float32) - posw
    t4 = posw + bp
    ta_ref[...] = jnp.concatenate([t1, t4], axis=1)
    tb_ref[...] = jnp.concatenate([t2, t3], axis=1)


def _make_tables(x, pos, W_src, W_dst, W_lin, W_pos, b_pos):
    n, d = x.shape
    blk = 1000
    nb = n // blk
    return pl.pallas_call(
        _tables_body,
        grid=(nb,),
        in_specs=[
            pl.BlockSpec((blk, d), lambda i: (i, 0)),
            pl.BlockSpec((blk, 3), lambda i: (i, 0)),
            pl.BlockSpec((d, d), lambda i: (0, 0)),
            pl.BlockSpec((d, d), lambda i: (0, 0)),
            pl.BlockSpec((d, d), lambda i: (0, 0)),
            pl.BlockSpec((3, d), lambda i: (0, 0)),
            pl.BlockSpec((1, d), lambda i: (0, 0)),
        ],
        out_specs=[pl.BlockSpec((blk, 2 * d), lambda i: (i, 0))] * 2,
        out_shape=[jax.ShapeDtypeStruct((n, 2 * d), jnp.float32)] * 2,
    )(x, pos, W_src, W_dst, W_lin, W_pos, b_pos.reshape(1, d))


def _alpha_sc_body(ta_hbm, tb_hbm, rows_hbm, cols_hbm, alpha_hbm, val_hbm,
                   rowb0, rowb1, colb0, colb1, tab0, tab1, tbb0, tbb1,
                   isem0, isem1, gsem0, gsem1, wsem0, wsem1):
    e_total = alpha_hbm.shape[0]
    nchunks = e_total // 80
    iters = nchunks // (NC * NS)
    cc = lax.axis_index("c")
    ss = lax.axis_index("s")
    wid = cc * NS + ss
    rowb = (rowb0, rowb1)
    colb = (colb0, colb1)
    tab = (tab0, tab1)
    tbb = (tbb0, tbb1)
    isem = (isem0, isem1)
    gsem = (gsem0, gsem1)
    wsem = (wsem0, wsem1)

    def e_of(k):
        return (wid + k * NC * NS) * 80

    def idx_copies(k, p):
        e0 = e_of(k)
        return (pltpu.make_async_copy(rows_hbm.at[pl.ds(e0, 80)], rowb[p],
                                      isem[p]),
                pltpu.make_async_copy(cols_hbm.at[pl.ds(e0, 80)], colb[p],
                                      isem[p]))

    def gather_copies(p):
        return (pltpu.make_async_copy(ta_hbm.at[rowb[p]], tab[p], gsem[p]),
                pltpu.make_async_copy(tb_hbm.at[colb[p]], tbb[p], gsem[p]))

    def write_copies(k, p):
        e0 = e_of(k)
        return (
            pltpu.make_async_copy(tab[p].at[pl.ds(0, 80), pl.ds(0, 128)],
                                  alpha_hbm.at[pl.ds(e0, 80)], wsem[p]),
            pltpu.make_async_copy(tab[p].at[pl.ds(0, 80), pl.ds(128, 128)],
                                  val_hbm.at[pl.ds(e0, 80)], wsem[p]))

    def compute(p):
        t, u = tab[p], tbb[p]

        def row_body(r, c2):
            for l in range(8):
                sl = pl.ds(l * 16, 16)
                t[r, sl] = t[r, sl] - u[r, sl]
            for l in range(8, 16):
                sl = pl.ds(l * 16, 16)
                t[r, sl] = t[r, sl] + u[r, sl]
            return c2
        lax.fori_loop(0, 80, row_body, 0)

    for d in idx_copies(0, 0):
        d.start()
    for d in idx_copies(1, 1):
        d.start()
    for d in idx_copies(0, 0):
        d.wait()
    for d in gather_copies(0):
        d.start()

    def step(k, p, k_is_first):
        p1 = 1 - p
        @pl.when(k + 1 < iters)
        def _():
            for d in idx_copies(k + 1, p1):
                d.wait()
            if not k_is_first:
                for d in write_copies(k - 1, p1):
                    d.wait()
            for d in gather_copies(p1):
                d.start()

        @pl.when(jnp.logical_and(k + 1 >= iters, k >= 1))
        def _():
            for d in write_copies(k - 1, p1):
                d.wait()
        for d in gather_copies(p):
            d.wait()
        @pl.when(k + 2 < iters)
        def _():
            for d in idx_copies(k + 2, p):
                d.start()
        compute(p)
        for d in write_copies(k, p):
            d.start()

    step(jnp.int32(0), 0, True)

    def pair(i, carry):
        step(2 * i + 1, 1, False)
        step(2 * i + 2, 0, False)
        return carry
    lax.fori_loop(0, (iters - 1) // 2, pair, 0)
    for d in write_copies(iters - 1, 0):
        d.wait()


def _alpha_gather(ta, tb, rows1d, cols1d, e_total):
    mesh = plsc.VectorSubcoreMesh(core_axis_name="c", subcore_axis_name="s")
    return pl.kernel(
        _alpha_sc_body,
        out_type=[
            jax.ShapeDtypeStruct((e_total, 128), jnp.float32),
            jax.ShapeDtypeStruct((e_total, 128), jnp.float32),
        ],
        mesh=mesh,
        scratch_types=(
            [pltpu.VMEM((80,), jnp.int32)] * 4 +
            [pltpu.VMEM((80, 256), jnp.float32)] * 4 +
            [pltpu.SemaphoreType.DMA] * 6
        ),
    )(ta, tb, rows1d, cols1d)


def _mlp_body(a_ref, wa1_ref, ba1_ref, wa2_ref, ba2_ref, a2_ref, mx_ref):
    h = jnp.maximum(
        jnp.dot(a_ref[...], wa1_ref[...],
                preferred_element_type=jnp.float32) + ba1_ref[...], 0.0)
    a2 = jnp.dot(h, wa2_ref[...],
                 preferred_element_type=jnp.float32) + ba2_ref[...]
    a2_ref[...] = a2
    mx_ref[...] = jnp.max(a2, axis=0, keepdims=True)[None]


def _edge_mlp(alpha, W_a1, b_a1, W_a2, b_a2):
    e_total, d = alpha.shape
    blk = 2000
    nb = e_total // blk
    a2, mx = pl.pallas_call(
        _mlp_body,
        grid=(nb,),
        in_specs=[
            pl.BlockSpec((blk, d), lambda i: (i, 0)),
            pl.BlockSpec((d, d), lambda i: (0, 0)),
            pl.BlockSpec((1, d), lambda i: (0, 0)),
            pl.BlockSpec((d, d), lambda i: (0, 0)),
            pl.BlockSpec((1, d), lambda i: (0, 0)),
        ],
        out_specs=[
            pl.BlockSpec((blk, d), lambda i: (i, 0)),
            pl.BlockSpec((1, 1, d), lambda i: (i, 0, 0)),
        ],
        out_shape=[
            jax.ShapeDtypeStruct((e_total, d), jnp.float32),
            jax.ShapeDtypeStruct((nb, 1, d), jnp.float32),
        ],
    )(alpha, W_a1, b_a1.reshape(1, d), W_a2, b_a2.reshape(1, d))
    return a2, mx


def _agg_sc_body(a2_hbm, val_hbm, rows_hbm, cvec_hbm, zeros_hbm,
                 s_hbm, o_hbm, rb0, rb1, a2b0, a2b1, vb0, vb1, cbuf,
                 acc_sp, lsem0, lsem1, ssem0, ssem1):
    n = s_hbm.shape[0]
    e_total = a2_hbm.shape[0]
    per_sub = e_total // NS
    iters = per_sub // 80
    cc = lax.axis_index("c")
    ss = lax.axis_index("s")
    rb = (rb0, rb1)
    a2b = (a2b0, a2b1)
    vb = (vb0, vb1)
    lsem = (lsem0, lsem1)
    ssem = (ssem0, ssem1)

    zbase = ((ss * (n // NS)) // 8) * 8
    pltpu.sync_copy(zeros_hbm, acc_sp.at[pl.ds(zbase, 632)])
    pltpu.sync_copy(cvec_hbm, cbuf)
    plsc.subcore_barrier()
    cv = cbuf[...]

    def load_copies(k, p):
        e0 = ss * per_sub + k * 80
        out = [pltpu.make_async_copy(rows_hbm.at[pl.ds(e0, 80)], rb[p],
                                     lsem[p]),
               pltpu.make_async_copy(a2_hbm.at[pl.ds(e0, 80)], a2b[p],
                                     lsem[p])]
        return out

    def val_copy(k, p):
        e0 = ss * per_sub + k * 80
        return pltpu.make_async_copy(val_hbm.at[pl.ds(e0, 80)], vb[p],
                                     lsem[p])

    def scat_copy(p, src):
        return pltpu.make_async_copy(src, acc_sp.at[rb[p]], ssem[p])

    def issue_loads(k, p):
        for d in load_copies(k, p):
            d.start()

        @pl.when(cc == 1)
        def _():
            val_copy(k, p).start()

    def wait_loads(k, p):
        for d in load_copies(k, p):
            d.wait()

        @pl.when(cc == 1)
        def _():
            val_copy(k, p).wait()

    def step(k, p, k_is_first):
        p1 = 1 - p
        if not k_is_first:
            @pl.when(cc == 0)
            def _():
                scat_copy(p1, a2b[p1]).wait()

            @pl.when(cc == 1)
            def _():
                scat_copy(p1, vb[p1]).wait()
        @pl.when(k + 1 < iters)
        def _():
            issue_loads(k + 1, p1)
        wait_loads(k, p)

        @pl.when(cc == 1)
        def _():
            def row_body1(r, c2):
                for l in range(8):
                    sl = pl.ds(l * 16, 16)
                    e = jnp.exp(a2b[p][r, sl] - cv)
                    vb[p][r, sl] = e * vb[p][r, sl]
                return c2
            lax.fori_loop(0, 80, row_body1, 0)
            scat_copy(p, vb[p]).start(add=True)

        @pl.when(cc == 0)
        def _():
            def row_body0(r, c2):
                for l in range(8):
                    sl = pl.ds(l * 16, 16)
                    a2b[p][r, sl] = jnp.exp(a2b[p][r, sl] - cv)
                return c2
            lax.fori_loop(0, 80, row_body0, 0)
            scat_copy(p, a2b[p]).start(add=True)

    issue_loads(jnp.int32(0), 0)
    step(jnp.int32(0), 0, True)

    def pair(i, carry):
        step(2 * i + 1, 1, False)
        step(2 * i + 2, 0, False)
        return carry
    lax.fori_loop(0, (iters - 2) // 2, pair, 0)
    step(jnp.int32(iters - 1), 1, False)
    @pl.when(cc == 0)
    def _():
        scat_copy(1, a2b[1]).wait()

    @pl.when(cc == 1)
    def _():
        scat_copy(1, vb[1]).wait()
    plsc.subcore_barrier()

    @pl.when(cc == 0)
    def _():
        pltpu.sync_copy(acc_sp.at[pl.ds(zbase, 632)],
                        s_hbm.at[pl.ds(zbase, 632)])

    @pl.when(cc == 1)
    def _():
        pltpu.sync_copy(acc_sp.at[pl.ds(zbase, 632)],
                        o_hbm.at[pl.ds(zbase, 632)])


def _softmax_agg(a2, val, rows1d, cvec, zeros, n):
    mesh = plsc.VectorSubcoreMesh(core_axis_name="c", subcore_axis_name="s")
    return pl.kernel(
        _agg_sc_body,
        out_type=[
            jax.ShapeDtypeStruct((n, 128), jnp.float32),
            jax.ShapeDtypeStruct((n, 128), jnp.float32),
        ],
        mesh=mesh,
        scratch_types=(
            [pltpu.VMEM((80,), jnp.int32)] * 2 +
            [pltpu.VMEM((80, 128), jnp.float32)] * 4 +
            [
                pltpu.VMEM((16,), jnp.float32),
                pltpu.VMEM_SHARED((n, 128), jnp.float32),
            ] +
            [pltpu.SemaphoreType.DMA] * 4
        ),
    )(a2, val, rows1d, cvec, zeros)


def _div_body(o_ref, s_ref, out_ref):
    out_ref[...] = o_ref[...] / (s_ref[...] + 1e-12)


def _divide(o_arr, s_arr):
    n, d = o_arr.shape
    blk = 1000
    return pl.pallas_call(
        _div_body,
        grid=(n // blk,),
        in_specs=[pl.BlockSpec((blk, d), lambda i: (i, 0))] * 2,
        out_specs=pl.BlockSpec((blk, d), lambda i: (i, 0)),
        out_shape=jax.ShapeDtypeStruct((n, d), jnp.float32),
    )(o_arr, s_arr)


def kernel(x, pos, edge_index, W_lin, W_src, W_dst, W_pos, b_pos,
           W_a1, b_a1, W_a2, b_a2):
    n = x.shape[0]
    e_total = edge_index.shape[0]
    rows = edge_index[:, 0]
    cols = edge_index[:, 1]

    ta, tb = _make_tables(x, pos, W_src, W_dst, W_lin, W_pos, b_pos)
    alpha, val = _alpha_gather(ta, tb, rows, cols, e_total)
    a2, mx = _edge_mlp(alpha, W_a1, b_a1, W_a2, b_a2)
    c = jnp.max(mx)
    cvec = jnp.full((16,), c, jnp.float32)
    zeros = jnp.zeros((632, 128), jnp.float32)
    s_arr, o_arr = _softmax_agg(a2, val, rows, cvec, zeros, n)
    return _divide(o_arr, s_arr)
